# Initial kernel scaffold; baseline (speedup 1.0000x reference)
import functools

import jax
import jax.numpy as jnp
from jax import lax
from jax.experimental import pallas as pl
from jax.experimental.pallas import tpu as pltpu
from jax.experimental.pallas import tpu_sc as plsc

N_NODES = 10000
N_EDGES = 320000
D = 128
L = 16
NC = 2
NS = 16
NW = NC * NS
ROWS = N_EDGES // 128
ROWS_PER_W = -(-ROWS // NW)
NODES_PER_TILE = N_NODES // NS
CW = 16

_sc_mesh = plsc.VectorSubcoreMesh(core_axis_name="c", subcore_axis_name="s")


@functools.partial(
    pl.kernel,
    out_type=(
        jax.ShapeDtypeStruct((NC, N_NODES, D), jnp.float32),
        jax.ShapeDtypeStruct((NC, N_NODES, CW), jnp.float32),
    ),
    mesh=_sc_mesh,
    scratch_types=[
        pltpu.VMEM((128, D), jnp.float32),
        pltpu.VMEM((128,), jnp.int32),
        pltpu.VMEM((128, CW), jnp.float32),
        pltpu.VMEM((128, D), jnp.float32),
        pltpu.VMEM((128, CW), jnp.float32),
        pltpu.VMEM_SHARED((N_NODES, D), jnp.float32),
        pltpu.VMEM_SHARED((N_NODES, CW), jnp.float32),
    ],
)
def _sc_scatter(edge_attr_hbm, src_hbm, sums_out, counts_out,
                rows_v, idx_v, ones_v, zrow_v, zcnt_v, sums_sh, counts_sh):
    c = lax.axis_index("c")
    s = lax.axis_index("s")
    w = s * NC + c

    zero16 = jnp.zeros((L,), jnp.float32)
    one16 = jnp.ones((L,), jnp.float32)

    def init_body(i, carry):
        for k in range(D // L):
            zrow_v[i, pl.ds(k * L, L)] = zero16
        zcnt_v[i, :] = zero16
        ones_v[i, :] = one16
        return carry

    lax.fori_loop(0, 128, init_body, 0)

    base = s * NODES_PER_TILE
    for k in range(5):
        pltpu.sync_copy(zrow_v.at[pl.ds(0, 125)],
                        sums_sh.at[pl.ds(base + k * 125, 125)])
        pltpu.sync_copy(zcnt_v.at[pl.ds(0, 125)],
                        counts_sh.at[pl.ds(base + k * 125, 125)])
    plsc.subcore_barrier()

    def body(i, carry):
        row = w * ROWS_PER_W + i

        @pl.when(row < ROWS)
        def _():
            pltpu.sync_copy(src_hbm.at[row], idx_v)
            pltpu.sync_copy(edge_attr_hbm.at[pl.ds(row * 128, 128)], rows_v)
            pltpu.sync_copy(rows_v, sums_sh.at[idx_v], add=True)
            pltpu.sync_copy(ones_v, counts_sh.at[idx_v], add=True)

        return carry

    lax.fori_loop(0, ROWS_PER_W, body, 0)
    plsc.subcore_barrier()

    pltpu.sync_copy(sums_sh.at[pl.ds(base, NODES_PER_TILE)],
                    sums_out.at[c, pl.ds(base, NODES_PER_TILE)])
    pltpu.sync_copy(counts_sh.at[pl.ds(base, NODES_PER_TILE)],
                    counts_out.at[c, pl.ds(base, NODES_PER_TILE)])


def _tc_body(na_ref, p0_ref, p1_ref, c0_ref, c1_ref, w_ref, b_ref, out_ref):
    cnt = (c0_ref[0] + c1_ref[0])[:, 0:1]
    mean = (p0_ref[0] + p1_ref[0]) / jnp.maximum(cnt, 1.0)
    x = na_ref[...] + mean * 0.5
    y = lax.dot_general(x, w_ref[...], (((1,), (1,)), ((), ())),
                        preferred_element_type=jnp.float32)
    out_ref[...] = jax.nn.sigmoid(y + b_ref[...])


_TC_B = 500


def _tc_dense(node_attr, sums, counts, W, b):
    grid = (N_NODES // _TC_B,)
    return pl.pallas_call(
        _tc_body,
        grid=grid,
        in_specs=[
            pl.BlockSpec((_TC_B, D), lambda i: (i, 0)),
            pl.BlockSpec((1, _TC_B, D), lambda i: (0, i, 0)),
            pl.BlockSpec((1, _TC_B, D), lambda i: (1, i, 0)),
            pl.BlockSpec((1, _TC_B, CW), lambda i: (0, i, 0)),
            pl.BlockSpec((1, _TC_B, CW), lambda i: (1, i, 0)),
            pl.BlockSpec((D, D), lambda i: (0, 0)),
            pl.BlockSpec((1, D), lambda i: (0, 0)),
        ],
        out_specs=pl.BlockSpec((_TC_B, D), lambda i: (i, 0)),
        out_shape=jax.ShapeDtypeStruct((N_NODES, D), jnp.float32),
    )(node_attr, sums, sums, counts, counts, W, b)


@functools.partial(
    pl.kernel,
    out_type=jax.ShapeDtypeStruct((N_EDGES, D), jnp.float32),
    mesh=_sc_mesh,
    scratch_types=[
        pltpu.VMEM((128,), jnp.int32),
        pltpu.VMEM((128,), jnp.int32),
        pltpu.VMEM((128, D), jnp.float32),
        pltpu.VMEM((128, D), jnp.float32),
        pltpu.SemaphoreType.DMA,
        pltpu.SemaphoreType.DMA,
    ],
)
def _sc_gather(emb_hbm, ei_hbm, out_hbm, idx_a, idx_b, buf_a, buf_b, sem_a, sem_b):
    c = lax.axis_index("c")
    s = lax.axis_index("s")
    w = s * NC + c

    def body(i, carry):
        row = w * ROWS_PER_W + i

        @pl.when(row < ROWS)
        def _():
            pltpu.sync_copy(ei_hbm.at[0, row], idx_a)
            pltpu.sync_copy(ei_hbm.at[1, row], idx_b)
            cp_a = pltpu.async_copy(emb_hbm.at[idx_a], buf_a, sem_a)
            cp_b = pltpu.async_copy(emb_hbm.at[idx_b], buf_b, sem_b)
            cp_a.wait()
            cp_b.wait()

            def vbody(r, carry2):
                for k in range(D // L):
                    sl = pl.ds(k * L, L)
                    buf_a[r, sl] = buf_a[r, sl] + buf_b[r, sl] * 0.5
                return carry2

            lax.fori_loop(0, 128, vbody, 0)
            pltpu.sync_copy(buf_a, out_hbm.at[pl.ds(row * 128, 128)])

        return carry

    lax.fori_loop(0, ROWS_PER_W, body, 0)


def kernel(node_attr, edge_attr, edge_index, W, b):
    src_rows = edge_index[0].reshape(ROWS, 128)
    ei_rows = edge_index.reshape(2, ROWS, 128)
    sums, counts = _sc_scatter(edge_attr, src_rows)
    emb = _tc_dense(node_attr, sums, counts, W, b.reshape(1, D))
    return _sc_gather(emb, ei_rows)


# speedup vs baseline: 3.7776x; 3.7776x over previous
import functools

import jax
import jax.numpy as jnp
from jax import lax
from jax.experimental import pallas as pl
from jax.experimental.pallas import tpu as pltpu
from jax.experimental.pallas import tpu_sc as plsc

N_NODES = 10000
N_EDGES = 320000
D = 128
L = 16
NC = 2
NS = 16
NW = NC * NS
ROWS = N_EDGES // 128
ROWS_PAD = 2560
ROWS_PER_T = ROWS_PAD // NS
G_PER_W = -(-ROWS // NW)
NSPLIT = 5000
TBL = 5120
SLAB = TBL // NS
CW = 16

_sc_mesh = plsc.VectorSubcoreMesh(core_axis_name="c", subcore_axis_name="s")


@functools.partial(
    pl.kernel,
    out_type=(
        jax.ShapeDtypeStruct((NC, TBL, D), jnp.float32),
        jax.ShapeDtypeStruct((NC, TBL, CW), jnp.float32),
    ),
    mesh=_sc_mesh,
    scratch_types=[
        pltpu.VMEM((128, D), jnp.float32),
        pltpu.VMEM((128,), jnp.int32),
        pltpu.VMEM((128, CW), jnp.float32),
        pltpu.VMEM((128, D), jnp.float32),
        pltpu.VMEM((128, CW), jnp.float32),
        pltpu.VMEM_SHARED((TBL, D), jnp.float32),
        pltpu.VMEM_SHARED((TBL, CW), jnp.float32),
        pltpu.SemaphoreType.DMA,
    ],
    compiler_params=pltpu.CompilerParams(use_tc_tiling_on_sc=False),
)
def _sc_scatter(edge_attr_hbm, src_hbm, sums_out, counts_out,
                rows_v, idx_v, ones_v, zrow_v, zcnt_v, sums_sh, counts_sh, sem):
    c = lax.axis_index("c")
    s = lax.axis_index("s")

    zero16 = jnp.zeros((L,), jnp.float32)
    one16 = jnp.ones((L,), jnp.float32)

    def init_body(i, carry):
        for k in range(D // L):
            zrow_v[i, pl.ds(k * L, L)] = zero16
        zcnt_v[i, :] = zero16
        ones_v[i, :] = one16
        return carry

    lax.fori_loop(0, 128, init_body, 0)

    base = s * SLAB
    pltpu.sync_copy(zrow_v, sums_sh.at[pl.ds(base, 128)])
    pltpu.sync_copy(zrow_v, sums_sh.at[pl.ds(base + 128, 128)])
    pltpu.sync_copy(zrow_v.at[pl.ds(0, 64)], sums_sh.at[pl.ds(base + 256, 64)])
    pltpu.sync_copy(zcnt_v, counts_sh.at[pl.ds(base, 128)])
    pltpu.sync_copy(zcnt_v, counts_sh.at[pl.ds(base + 128, 128)])
    pltpu.sync_copy(zcnt_v.at[pl.ds(0, 64)], counts_sh.at[pl.ds(base + 256, 64)])
    plsc.subcore_barrier()

    for i in range(ROWS_PER_T):
        row = s * ROWS_PER_T + i
        rowc = jnp.minimum(row, ROWS - 1)
        pltpu.async_copy(
            src_hbm.at[pl.ds((c * ROWS_PAD + row) * 128, 128)], idx_v, sem
        ).wait()
        pltpu.async_copy(
            edge_attr_hbm.at[pl.ds(rowc * 128, 128)], rows_v, sem
        ).wait()
        pltpu.async_copy(rows_v, sums_sh.at[idx_v], sem, add=True).wait()
        pltpu.async_copy(ones_v, counts_sh.at[idx_v], sem, add=True).wait()

    plsc.subcore_barrier()

    pltpu.sync_copy(sums_sh.at[pl.ds(base, SLAB)],
                    sums_out.at[c, pl.ds(base, SLAB)])
    pltpu.sync_copy(counts_sh.at[pl.ds(base, SLAB)],
                    counts_out.at[c, pl.ds(base, SLAB)])


def _tc_body(na_ref, p_ref, c_ref, w_ref, b_ref, out_ref):
    cnt = c_ref[0][:, 0:1]
    mean = p_ref[0] / jnp.maximum(cnt, 1.0)
    x = na_ref[...] + mean * 0.5
    y = lax.dot_general(x, w_ref[...], (((1,), (1,)), ((), ())),
                        preferred_element_type=jnp.float32)
    out_ref[...] = jax.nn.sigmoid(y + b_ref[...])


_TC_B = 1000


def _tc_dense(node_attr, sums, counts, W, b):
    grid = (N_NODES // _TC_B,)
    nb = NSPLIT // _TC_B
    return pl.pallas_call(
        _tc_body,
        grid=grid,
        in_specs=[
            pl.BlockSpec((_TC_B, D), lambda i: (i, 0)),
            pl.BlockSpec((1, _TC_B, D), lambda i: (i // nb, i % nb, 0)),
            pl.BlockSpec((1, _TC_B, CW), lambda i: (i // nb, i % nb, 0)),
            pl.BlockSpec((D, D), lambda i: (0, 0)),
            pl.BlockSpec((1, D), lambda i: (0, 0)),
        ],
        out_specs=pl.BlockSpec((_TC_B, D), lambda i: (i, 0)),
        out_shape=jax.ShapeDtypeStruct((N_NODES, D), jnp.float32),
    )(node_attr, sums, counts, W, b)


GR = 3
GCH = GR * 128
G_STEPS = -(-G_PER_W // GR)


@functools.partial(
    pl.kernel,
    out_type=jax.ShapeDtypeStruct((N_EDGES, D), jnp.float32),
    mesh=_sc_mesh,
    scratch_types=[
        pltpu.VMEM((GCH,), jnp.int32),
        pltpu.VMEM((GCH,), jnp.int32),
        pltpu.VMEM((GCH, D), jnp.float32),
        pltpu.VMEM((GCH, D), jnp.float32),
        pltpu.SemaphoreType.DMA,
        pltpu.SemaphoreType.DMA,
    ],
    compiler_params=pltpu.CompilerParams(use_tc_tiling_on_sc=False),
)
def _sc_gather(emb_hbm, src_hbm, dst_hbm, out_hbm,
               idx_a, idx_b, buf_a, buf_b, sem_a, sem_b):
    c = lax.axis_index("c")
    s = lax.axis_index("s")
    w = s * NC + c

    def vbody(r, carry2):
        for k in range(D // L):
            sl = pl.ds(k * L, L)
            buf_a[r, sl] = buf_a[r, sl] + buf_b[r, sl] * 0.5
        return carry2

    for j in range(G_STEPS):
        base = jnp.minimum(w * G_PER_W + j * GR, ROWS - GR)
        cp_i = pltpu.async_copy(src_hbm.at[pl.ds(base * 128, GCH)], idx_a, sem_a)
        cp_j = pltpu.async_copy(dst_hbm.at[pl.ds(base * 128, GCH)], idx_b, sem_b)
        cp_i.wait()
        cp_j.wait()
        cp_a = pltpu.async_copy(emb_hbm.at[idx_a], buf_a, sem_a)
        cp_b = pltpu.async_copy(emb_hbm.at[idx_b], buf_b, sem_b)
        cp_a.wait()
        cp_b.wait()
        lax.fori_loop(0, GCH, vbody, 0)
        pltpu.async_copy(buf_a, out_hbm.at[pl.ds(base * 128, GCH)], sem_a).wait()


def _core_local_indices(src):
    spread = (jnp.arange(N_EDGES, dtype=jnp.int32) % L) * 7
    junk = NSPLIT + spread
    loc0 = jnp.where(src < NSPLIT, src, junk)
    loc1 = jnp.where(src >= NSPLIT, src - NSPLIT, junk)
    pad = NSPLIT + (jnp.arange(ROWS_PAD * 128 - N_EDGES, dtype=jnp.int32) % L) * 7
    return jnp.concatenate([loc0, pad, loc1, pad])


def kernel(node_attr, edge_attr, edge_index, W, b):
    src = edge_index[0]
    dst = edge_index[1]
    sums, counts = _sc_scatter(edge_attr, _core_local_indices(src))
    emb = _tc_dense(node_attr, sums, counts, W, b.reshape(1, D))
    return _sc_gather(emb, src, dst)


# speedup vs baseline: 5.5505x; 1.4693x over previous
import functools

import jax
import jax.numpy as jnp
from jax import lax
from jax.experimental import pallas as pl
from jax.experimental.pallas import tpu as pltpu
from jax.experimental.pallas import tpu_sc as plsc

N_NODES = 10000
N_EDGES = 320000
D = 128
L = 16
NC = 2
NS = 16
NW = NC * NS
ROWS = N_EDGES // 128
ROWS_PAD = 2560
ROWS_PER_T = ROWS_PAD // NS
G_PER_W = -(-ROWS // NW)
NSPLIT = 5000
TBL = 5120
SLAB = TBL // NS
CW = 16

_sc_mesh = plsc.VectorSubcoreMesh(core_axis_name="c", subcore_axis_name="s")


@functools.partial(
    pl.kernel,
    out_type=(
        jax.ShapeDtypeStruct((NC, TBL, D), jnp.float32),
        jax.ShapeDtypeStruct((NC, TBL, CW), jnp.float32),
    ),
    mesh=_sc_mesh,
    scratch_types=[
        pltpu.VMEM((128, D), jnp.float32),
        pltpu.VMEM((128, D), jnp.float32),
        pltpu.VMEM((128,), jnp.int32),
        pltpu.VMEM((128,), jnp.int32),
        pltpu.VMEM((128, CW), jnp.float32),
        pltpu.VMEM((128, D), jnp.float32),
        pltpu.VMEM((128, CW), jnp.float32),
        pltpu.VMEM_SHARED((TBL, D), jnp.float32),
        pltpu.VMEM_SHARED((TBL, CW), jnp.float32),
        pltpu.SemaphoreType.DMA,
        pltpu.SemaphoreType.DMA,
        pltpu.SemaphoreType.DMA,
        pltpu.SemaphoreType.DMA,
    ],
    compiler_params=pltpu.CompilerParams(use_tc_tiling_on_sc=False),
)
def _sc_scatter(edge_attr_hbm, src_hbm, sums_out, counts_out,
                rows_v0, rows_v1, idx_v0, idx_v1, ones_v, zrow_v, zcnt_v,
                sums_sh, counts_sh, sem_r0, sem_r1, sem_w0, sem_w1):
    c = lax.axis_index("c")
    s = lax.axis_index("s")

    zero16 = jnp.zeros((L,), jnp.float32)
    one16 = jnp.ones((L,), jnp.float32)

    def init_body(i, carry):
        for k in range(D // L):
            zrow_v[i, pl.ds(k * L, L)] = zero16
        zcnt_v[i, :] = zero16
        ones_v[i, :] = one16
        return carry

    lax.fori_loop(0, 128, init_body, 0)

    base = s * SLAB
    pltpu.sync_copy(zrow_v, sums_sh.at[pl.ds(base, 128)])
    pltpu.sync_copy(zrow_v, sums_sh.at[pl.ds(base + 128, 128)])
    pltpu.sync_copy(zrow_v.at[pl.ds(0, 64)], sums_sh.at[pl.ds(base + 256, 64)])
    pltpu.sync_copy(zcnt_v, counts_sh.at[pl.ds(base, 128)])
    pltpu.sync_copy(zcnt_v, counts_sh.at[pl.ds(base + 128, 128)])
    pltpu.sync_copy(zcnt_v.at[pl.ds(0, 64)], counts_sh.at[pl.ds(base + 256, 64)])
    plsc.subcore_barrier()

    rows_b = (rows_v0, rows_v1)
    idx_b = (idx_v0, idx_v1)
    sem_r = (sem_r0, sem_r1)
    sem_w = (sem_w0, sem_w1)
    pend_reads = {}
    pend_writes = {}

    def start_reads(i):
        slot = i & 1
        row = s * ROWS_PER_T + i
        rowc = jnp.minimum(row, ROWS - 1)
        d1 = pltpu.async_copy(
            src_hbm.at[pl.ds((c * ROWS_PAD + row) * 128, 128)],
            idx_b[slot], sem_r[slot])
        d2 = pltpu.async_copy(
            edge_attr_hbm.at[pl.ds(rowc * 128, 128)], rows_b[slot], sem_r[slot])
        pend_reads[i] = (d1, d2)

    start_reads(0)
    for i in range(ROWS_PER_T):
        slot = i & 1
        if i >= 1:
            for dsc in pend_writes.pop(i - 1):
                dsc.wait()
        if i + 1 < ROWS_PER_T:
            start_reads(i + 1)
        for dsc in pend_reads.pop(i):
            dsc.wait()
        dw1 = pltpu.async_copy(rows_b[slot], sums_sh.at[idx_b[slot]],
                               sem_w[slot], add=True)
        dw2 = pltpu.async_copy(ones_v, counts_sh.at[idx_b[slot]],
                               sem_w[slot], add=True)
        pend_writes[i] = (dw1, dw2)
    for dsc in pend_writes.pop(ROWS_PER_T - 1):
        dsc.wait()

    plsc.subcore_barrier()

    pltpu.sync_copy(sums_sh.at[pl.ds(base, SLAB)],
                    sums_out.at[c, pl.ds(base, SLAB)])
    pltpu.sync_copy(counts_sh.at[pl.ds(base, SLAB)],
                    counts_out.at[c, pl.ds(base, SLAB)])


def _tc_body(na_ref, p_ref, c_ref, w_ref, b_ref, out_ref):
    cnt = c_ref[0][:, 0:1]
    mean = p_ref[0] / jnp.maximum(cnt, 1.0)
    x = na_ref[...] + mean * 0.5
    y = lax.dot_general(x, w_ref[...], (((1,), (1,)), ((), ())),
                        preferred_element_type=jnp.float32)
    out_ref[...] = jax.nn.sigmoid(y + b_ref[...])


_TC_B = 1000


def _tc_dense(node_attr, sums, counts, W, b):
    grid = (N_NODES // _TC_B,)
    nb = NSPLIT // _TC_B
    return pl.pallas_call(
        _tc_body,
        grid=grid,
        in_specs=[
            pl.BlockSpec((_TC_B, D), lambda i: (i, 0)),
            pl.BlockSpec((1, _TC_B, D), lambda i: (i // nb, i % nb, 0)),
            pl.BlockSpec((1, _TC_B, CW), lambda i: (i // nb, i % nb, 0)),
            pl.BlockSpec((D, D), lambda i: (0, 0)),
            pl.BlockSpec((1, D), lambda i: (0, 0)),
        ],
        out_specs=pl.BlockSpec((_TC_B, D), lambda i: (i, 0)),
        out_shape=jax.ShapeDtypeStruct((N_NODES, D), jnp.float32),
    )(node_attr, sums, counts, W, b)


GR = 2
GCH = GR * 128
G_STEPS = -(-G_PER_W // GR)


@functools.partial(
    pl.kernel,
    out_type=jax.ShapeDtypeStruct((N_EDGES, D), jnp.float32),
    mesh=_sc_mesh,
    scratch_types=[
        pltpu.VMEM((GCH,), jnp.int32),
        pltpu.VMEM((GCH,), jnp.int32),
        pltpu.VMEM((GCH,), jnp.int32),
        pltpu.VMEM((GCH,), jnp.int32),
        pltpu.VMEM((GCH, D), jnp.float32),
        pltpu.VMEM((GCH, D), jnp.float32),
        pltpu.VMEM((GCH, D), jnp.float32),
        pltpu.SemaphoreType.DMA,
        pltpu.SemaphoreType.DMA,
        pltpu.SemaphoreType.DMA,
        pltpu.SemaphoreType.DMA,
    ],
    compiler_params=pltpu.CompilerParams(use_tc_tiling_on_sc=False),
)
def _sc_gather(emb_hbm, src_hbm, dst_hbm, out_hbm,
               ia0, ia1, ib0, ib1, g_a, g_b, outb,
               sem_i0, sem_i1, sem_g, sem_o):
    c = lax.axis_index("c")
    s = lax.axis_index("s")
    w = s * NC + c

    ia = (ia0, ia1)
    ib = (ib0, ib1)
    sem_i = (sem_i0, sem_i1)

    def vbody(r, carry2):
        for k in range(D // L):
            sl = pl.ds(k * L, L)
            outb[r, sl] = g_a[r, sl] + g_b[r, sl] * 0.5
        return carry2

    def base_of(j):
        return jnp.minimum(w * G_PER_W + j * GR, ROWS - GR)

    pend_idx = {}
    pend_g = {}

    def start_idx(j):
        slot = j & 1
        base = base_of(j)
        d1 = pltpu.async_copy(src_hbm.at[pl.ds(base * 128, GCH)],
                              ia[slot], sem_i[slot])
        d2 = pltpu.async_copy(dst_hbm.at[pl.ds(base * 128, GCH)],
                              ib[slot], sem_i[slot])
        pend_idx[j] = (d1, d2)

    def start_gathers(j):
        slot = j & 1
        d1 = pltpu.async_copy(emb_hbm.at[ia[slot]], g_a, sem_g)
        d2 = pltpu.async_copy(emb_hbm.at[ib[slot]], g_b, sem_g)
        pend_g[j] = (d1, d2)

    start_idx(0)
    for dsc in pend_idx.pop(0):
        dsc.wait()
    start_gathers(0)
    pend_write = None
    for j in range(G_STEPS):
        if j + 1 < G_STEPS:
            start_idx(j + 1)
        for dsc in pend_g.pop(j):
            dsc.wait()
        if pend_write is not None:
            pend_write.wait()
        lax.fori_loop(0, GCH, vbody, 0)
        if j + 1 < G_STEPS:
            for dsc in pend_idx.pop(j + 1):
                dsc.wait()
            start_gathers(j + 1)
        pend_write = pltpu.async_copy(
            outb, out_hbm.at[pl.ds(base_of(j) * 128, GCH)], sem_o)
    pend_write.wait()


def _core_local_indices(src):
    spread = (jnp.arange(N_EDGES, dtype=jnp.int32) % L) * 7
    junk = NSPLIT + spread
    loc0 = jnp.where(src < NSPLIT, src, junk)
    loc1 = jnp.where(src >= NSPLIT, src - NSPLIT, junk)
    pad = NSPLIT + (jnp.arange(ROWS_PAD * 128 - N_EDGES, dtype=jnp.int32) % L) * 7
    return jnp.concatenate([loc0, pad, loc1, pad])


def kernel(node_attr, edge_attr, edge_index, W, b):
    src = edge_index[0]
    dst = edge_index[1]
    sums, counts = _sc_scatter(edge_attr, _core_local_indices(src))
    emb = _tc_dense(node_attr, sums, counts, W, b.reshape(1, D))
    return _sc_gather(emb, src, dst)


# speedup vs baseline: 6.7094x; 1.2088x over previous
import functools

import jax
import jax.numpy as jnp
from jax import lax
from jax.experimental import pallas as pl
from jax.experimental.pallas import tpu as pltpu
from jax.experimental.pallas import tpu_sc as plsc

N_NODES = 10000
N_EDGES = 320000
D = 128
L = 16
NC = 2
NS = 16
NW = NC * NS
ROWS = N_EDGES // 128
ROWS_PAD = 2560
ROWS_PER_T = ROWS_PAD // NS
G_PER_W = -(-ROWS // NW)
NSPLIT = 5000
TBL = 5120
SLAB = TBL // NS
CW = 16

_sc_mesh = plsc.VectorSubcoreMesh(core_axis_name="c", subcore_axis_name="s")


@functools.partial(
    pl.kernel,
    out_type=(
        jax.ShapeDtypeStruct((NC, TBL, D), jnp.float32),
        jax.ShapeDtypeStruct((NC, TBL, CW), jnp.float32),
    ),
    mesh=_sc_mesh,
    scratch_types=[
        pltpu.VMEM((128, D), jnp.float32),
        pltpu.VMEM((128, D), jnp.float32),
        pltpu.VMEM((128,), jnp.int32),
        pltpu.VMEM((128,), jnp.int32),
        pltpu.VMEM((128, CW), jnp.float32),
        pltpu.VMEM((128, D), jnp.float32),
        pltpu.VMEM((128, CW), jnp.float32),
        pltpu.VMEM_SHARED((TBL, D), jnp.float32),
        pltpu.VMEM_SHARED((TBL, CW), jnp.float32),
        pltpu.SemaphoreType.DMA,
        pltpu.SemaphoreType.DMA,
        pltpu.SemaphoreType.DMA,
        pltpu.SemaphoreType.DMA,
    ],
    compiler_params=pltpu.CompilerParams(use_tc_tiling_on_sc=False),
)
def _sc_scatter(edge_attr_hbm, src_hbm, sums_out, counts_out,
                rows_v0, rows_v1, idx_v0, idx_v1, ones_v, zrow_v, zcnt_v,
                sums_sh, counts_sh, sem_r0, sem_r1, sem_w0, sem_w1):
    c = lax.axis_index("c")
    s = lax.axis_index("s")

    zero16 = jnp.zeros((L,), jnp.float32)
    one16 = jnp.ones((L,), jnp.float32)

    def init_body(i, carry):
        for k in range(D // L):
            zrow_v[i, pl.ds(k * L, L)] = zero16
        zcnt_v[i, :] = zero16
        ones_v[i, :] = one16
        return carry

    lax.fori_loop(0, 128, init_body, 0)

    base = s * SLAB
    pltpu.sync_copy(zrow_v, sums_sh.at[pl.ds(base, 128)])
    pltpu.sync_copy(zrow_v, sums_sh.at[pl.ds(base + 128, 128)])
    pltpu.sync_copy(zrow_v.at[pl.ds(0, 64)], sums_sh.at[pl.ds(base + 256, 64)])
    pltpu.sync_copy(zcnt_v, counts_sh.at[pl.ds(base, 128)])
    pltpu.sync_copy(zcnt_v, counts_sh.at[pl.ds(base + 128, 128)])
    pltpu.sync_copy(zcnt_v.at[pl.ds(0, 64)], counts_sh.at[pl.ds(base + 256, 64)])
    plsc.subcore_barrier()

    rows_b = (rows_v0, rows_v1)
    idx_b = (idx_v0, idx_v1)
    sem_r = (sem_r0, sem_r1)
    sem_w = (sem_w0, sem_w1)
    pend_reads = {}
    pend_writes = {}

    def start_reads(i):
        slot = i & 1
        row = s * ROWS_PER_T + i
        rowc = jnp.minimum(row, ROWS - 1)
        d1 = pltpu.async_copy(
            src_hbm.at[pl.ds((c * ROWS_PAD + row) * 128, 128)],
            idx_b[slot], sem_r[slot])
        d2 = pltpu.async_copy(
            edge_attr_hbm.at[pl.ds(rowc * 128, 128)], rows_b[slot], sem_r[slot])
        pend_reads[i] = (d1, d2)

    start_reads(0)
    for i in range(ROWS_PER_T):
        slot = i & 1
        if i >= 1:
            for dsc in pend_writes.pop(i - 1):
                dsc.wait()
        if i + 1 < ROWS_PER_T:
            start_reads(i + 1)
        for dsc in pend_reads.pop(i):
            dsc.wait()
        dw1 = pltpu.async_copy(rows_b[slot], sums_sh.at[idx_b[slot]],
                               sem_w[slot], add=True)
        dw2 = pltpu.async_copy(ones_v, counts_sh.at[idx_b[slot]],
                               sem_w[slot], add=True)
        pend_writes[i] = (dw1, dw2)
    for dsc in pend_writes.pop(ROWS_PER_T - 1):
        dsc.wait()

    plsc.subcore_barrier()

    pltpu.sync_copy(sums_sh.at[pl.ds(base, SLAB)],
                    sums_out.at[c, pl.ds(base, SLAB)])
    pltpu.sync_copy(counts_sh.at[pl.ds(base, SLAB)],
                    counts_out.at[c, pl.ds(base, SLAB)])


def _tc_body(na_ref, p_ref, c_ref, w_ref, b_ref, out_ref):
    cnt = c_ref[0][:, 0:1]
    mean = p_ref[0] / jnp.maximum(cnt, 1.0)
    x = na_ref[...] + mean * 0.5
    y = lax.dot_general(x, w_ref[...], (((1,), (1,)), ((), ())),
                        preferred_element_type=jnp.float32)
    out_ref[...] = jax.nn.sigmoid(y + b_ref[...])


_TC_B = 1000


def _tc_dense(node_attr, sums, counts, W, b):
    grid = (N_NODES // _TC_B,)
    nb = NSPLIT // _TC_B
    return pl.pallas_call(
        _tc_body,
        grid=grid,
        in_specs=[
            pl.BlockSpec((_TC_B, D), lambda i: (i, 0)),
            pl.BlockSpec((1, _TC_B, D), lambda i: (i // nb, i % nb, 0)),
            pl.BlockSpec((1, _TC_B, CW), lambda i: (i // nb, i % nb, 0)),
            pl.BlockSpec((D, D), lambda i: (0, 0)),
            pl.BlockSpec((1, D), lambda i: (0, 0)),
        ],
        out_specs=pl.BlockSpec((_TC_B, D), lambda i: (i, 0)),
        out_shape=jax.ShapeDtypeStruct((N_NODES, D), jnp.float32),
    )(node_attr, sums, counts, W, b)


GCH = 240
E_PER_W = N_EDGES // NW
G_STEPS = -(-E_PER_W // GCH)


@functools.partial(
    pl.kernel,
    out_type=jax.ShapeDtypeStruct((N_EDGES, D), jnp.float32),
    mesh=_sc_mesh,
    scratch_types=[
        pltpu.VMEM((GCH,), jnp.int32),
        pltpu.VMEM((GCH,), jnp.int32),
        pltpu.VMEM((GCH,), jnp.int32),
        pltpu.VMEM((GCH,), jnp.int32),
        pltpu.VMEM((GCH, D), jnp.float32),
        pltpu.VMEM((GCH, D), jnp.float32),
        pltpu.VMEM((GCH, D), jnp.float32),
        pltpu.VMEM((GCH, D), jnp.float32),
        pltpu.SemaphoreType.DMA,
        pltpu.SemaphoreType.DMA,
        pltpu.SemaphoreType.DMA,
        pltpu.SemaphoreType.DMA,
        pltpu.SemaphoreType.DMA,
        pltpu.SemaphoreType.DMA,
    ],
    compiler_params=pltpu.CompilerParams(use_tc_tiling_on_sc=False),
)
def _sc_gather(emb_hbm, src_hbm, dst_hbm, out_hbm,
               ia0, ia1, ib0, ib1, ga0, ga1, gb0, gb1,
               sem_i0, sem_i1, sem_g0, sem_g1, sem_o0, sem_o1):
    c = lax.axis_index("c")
    s = lax.axis_index("s")
    w = s * NC + c

    ia = (ia0, ia1)
    ib = (ib0, ib1)
    ga = (ga0, ga1)
    gb = (gb0, gb1)
    sem_i = (sem_i0, sem_i1)
    sem_g = (sem_g0, sem_g1)
    sem_o = (sem_o0, sem_o1)

    def make_vbody(slot):
        def vbody(r, carry2):
            for k in range(D // L):
                sl = pl.ds(k * L, L)
                ga[slot][r, sl] = ga[slot][r, sl] + gb[slot][r, sl] * 0.5
            return carry2
        return vbody

    vbodies = (make_vbody(0), make_vbody(1))

    def base_of(j):
        return jnp.minimum(w * E_PER_W + j * GCH, N_EDGES - GCH)

    pend_idx = {}
    pend_g = {}
    pend_w = {}

    def start_idx(j):
        slot = j & 1
        base = base_of(j)
        d1 = pltpu.async_copy(src_hbm.at[pl.ds(base, GCH)], ia[slot],
                              sem_i[slot])
        d2 = pltpu.async_copy(dst_hbm.at[pl.ds(base, GCH)], ib[slot],
                              sem_i[slot])
        pend_idx[j] = (d1, d2)

    def start_gathers(j):
        slot = j & 1
        d1 = pltpu.async_copy(emb_hbm.at[ia[slot]], ga[slot], sem_g[slot])
        d2 = pltpu.async_copy(emb_hbm.at[ib[slot]], gb[slot], sem_g[slot])
        pend_g[j] = (d1, d2)

    start_idx(0)
    start_idx(1)
    for dsc in pend_idx.pop(0):
        dsc.wait()
    start_gathers(0)
    for j in range(G_STEPS):
        slot = j & 1
        for dsc in pend_g.pop(j):
            dsc.wait()
        if j + 2 < G_STEPS:
            start_idx(j + 2)
        if j + 1 < G_STEPS:
            for dsc in pend_idx.pop(j + 1):
                dsc.wait()
            if j - 1 in pend_w:
                pend_w.pop(j - 1).wait()
            start_gathers(j + 1)
        lax.fori_loop(0, GCH, vbodies[slot], 0)
        pend_w[j] = pltpu.async_copy(
            ga[slot], out_hbm.at[pl.ds(base_of(j), GCH)], sem_o[slot])
    for jj in sorted(pend_w):
        pend_w.pop(jj).wait()


def _core_local_indices(src):
    spread = (jnp.arange(N_EDGES, dtype=jnp.int32) % L) * 7
    junk = NSPLIT + spread
    loc0 = jnp.where(src < NSPLIT, src, junk)
    loc1 = jnp.where(src >= NSPLIT, src - NSPLIT, junk)
    pad = NSPLIT + (jnp.arange(ROWS_PAD * 128 - N_EDGES, dtype=jnp.int32) % L) * 7
    return jnp.concatenate([loc0, pad, loc1, pad])


def kernel(node_attr, edge_attr, edge_index, W, b):
    src = edge_index[0]
    dst = edge_index[1]
    sums, counts = _sc_scatter(edge_attr, _core_local_indices(src))
    emb = _tc_dense(node_attr, sums, counts, W, b.reshape(1, D))
    return _sc_gather(emb, src, dst)
